# initial kernel scaffold (unmeasured)
import jax
import jax.numpy as jnp
from jax import lax
from jax.experimental import pallas as pl
from jax.experimental.pallas import tpu as pltpu

T = 8


def kernel(A, B):
    M, K = A.shape
    _, N = B.shape
    TN = N // T

    a = A.astype(jnp.bfloat16)
    b = B.astype(jnp.bfloat16)

    def body(a_ref, b_ref, out_ref, ptheirs_ref, work_ref,
             send_sems, recv_sems, copy_sems):
        my_x = lax.axis_index("x")
        my_y = lax.axis_index("y")
        peer = (1 - my_x, my_y)

        barrier = pltpu.get_barrier_semaphore()
        pl.semaphore_signal(barrier, inc=1, device_id=peer,
                            device_id_type=pl.DeviceIdType.MESH)
        pl.semaphore_wait(barrier, 1)

        for t in range(T):
            sl = pl.ds(t * TN, TN)
            work_ref[0] = jnp.dot(
                a_ref[...], b_ref[:, sl],
                preferred_element_type=jnp.float32,
            ).astype(jnp.bfloat16)
            rdma = pltpu.make_async_remote_copy(
                src_ref=work_ref.at[0],
                dst_ref=ptheirs_ref.at[t],
                send_sem=send_sems.at[t],
                recv_sem=recv_sems.at[t],
                device_id=peer,
                device_id_type=pl.DeviceIdType.MESH,
            )
            rdma.start()
            rdma.wait()
            load = pltpu.make_async_copy(
                ptheirs_ref.at[t], work_ref.at[1], copy_sems.at[0])
            load.start()
            load.wait()
            work_ref[2] = (
                work_ref[0].astype(jnp.float32)
                + work_ref[1].astype(jnp.float32)
            ).astype(jnp.bfloat16)
            store = pltpu.make_async_copy(
                work_ref.at[2], out_ref.at[:, sl], copy_sems.at[1])
            store.start()
            store.wait()

    out, _ = pl.pallas_call(
        body,
        out_shape=[
            jax.ShapeDtypeStruct((M, N), jnp.bfloat16),
            jax.ShapeDtypeStruct((T, M, TN), jnp.bfloat16),
        ],
        in_specs=[
            pl.BlockSpec(memory_space=pltpu.MemorySpace.VMEM),
            pl.BlockSpec(memory_space=pltpu.MemorySpace.VMEM),
        ],
        out_specs=[
            pl.BlockSpec(memory_space=pl.ANY),
            pl.BlockSpec(memory_space=pl.ANY),
        ],
        scratch_shapes=[
            pltpu.VMEM((3, M, TN), jnp.bfloat16),
            pltpu.SemaphoreType.DMA((T,)),
            pltpu.SemaphoreType.DMA((T,)),
            pltpu.SemaphoreType.DMA((2,)),
        ],
        compiler_params=pltpu.CompilerParams(collective_id=0),
    )(a, b)
    return out


# baseline (device time: 560067 ns/iter reference)
import jax
import jax.numpy as jnp
from jax import lax
from jax.experimental import pallas as pl
from jax.experimental.pallas import tpu as pltpu

T = 8


def kernel(A, B):
    M, K = A.shape
    _, N = B.shape
    TN = N // T

    a = A.astype(jnp.bfloat16)
    b = B.astype(jnp.bfloat16)

    def body(a_ref, b_ref, out_ref, ptheirs_ref, work_ref,
             send_sems, recv_sems, copy_sems):
        my_x = lax.axis_index("x")
        my_y = lax.axis_index("y")
        peer = (1 - my_x, my_y)

        barrier = pltpu.get_barrier_semaphore()
        pl.semaphore_signal(barrier, inc=1, device_id=peer,
                            device_id_type=pl.DeviceIdType.MESH)
        pl.semaphore_wait(barrier, 1)

        def step(t, carry):
            sl = pl.ds(t * TN, TN)
            work_ref[0] = jnp.dot(
                a_ref[...], b_ref[:, sl],
                preferred_element_type=jnp.float32,
            ).astype(jnp.bfloat16)
            rdma = pltpu.make_async_remote_copy(
                src_ref=work_ref.at[0],
                dst_ref=ptheirs_ref.at[t],
                send_sem=send_sems.at[t],
                recv_sem=recv_sems.at[t],
                device_id=peer,
                device_id_type=pl.DeviceIdType.MESH,
            )
            rdma.start()
            rdma.wait()
            load = pltpu.make_async_copy(
                ptheirs_ref.at[t], work_ref.at[1], copy_sems.at[0])
            load.start()
            load.wait()
            work_ref[2] = (
                work_ref[0].astype(jnp.float32)
                + work_ref[1].astype(jnp.float32)
            ).astype(jnp.bfloat16)
            store = pltpu.make_async_copy(
                work_ref.at[2], out_ref.at[:, sl], copy_sems.at[1])
            store.start()
            store.wait()
            return carry

        lax.fori_loop(0, T, step, 0)

    out, _ = pl.pallas_call(
        body,
        out_shape=[
            jax.ShapeDtypeStruct((M, N), jnp.bfloat16),
            jax.ShapeDtypeStruct((T, M, TN), jnp.bfloat16),
        ],
        in_specs=[
            pl.BlockSpec(memory_space=pltpu.MemorySpace.VMEM),
            pl.BlockSpec(memory_space=pltpu.MemorySpace.VMEM),
        ],
        out_specs=[
            pl.BlockSpec(memory_space=pl.ANY),
            pl.BlockSpec(memory_space=pl.ANY),
        ],
        scratch_shapes=[
            pltpu.VMEM((3, M, TN), jnp.bfloat16),
            pltpu.SemaphoreType.DMA((T,)),
            pltpu.SemaphoreType.DMA((T,)),
            pltpu.SemaphoreType.DMA((2,)),
        ],
        compiler_params=pltpu.CompilerParams(
            collective_id=0,
            vmem_limit_bytes=64 * 1024 * 1024,
        ),
    )(a, b)
    return out


# device time: 553913 ns/iter; 1.0111x vs baseline; 1.0111x over previous
import jax
import jax.numpy as jnp
from jax import lax
from jax.experimental import pallas as pl
from jax.experimental.pallas import tpu as pltpu

T = 8


def kernel(A, B):
    M, K = A.shape
    _, N = B.shape
    TN = N // T

    a = A.astype(jnp.bfloat16)
    b = B.astype(jnp.bfloat16)

    def body(a_ref, b_ref, out_ref, ptheirs_ref, work_ref,
             send_sems, recv_sems, copy_sems):
        my_x = lax.axis_index("x")
        my_y = lax.axis_index("y")
        peer = (1 - my_x, my_y)

        barrier = pltpu.get_barrier_semaphore()
        pl.semaphore_signal(barrier, inc=1, device_id=peer,
                            device_id_type=pl.DeviceIdType.MESH)
        pl.semaphore_wait(barrier, 1)

        def rdma_for(t, slot):
            return pltpu.make_async_remote_copy(
                src_ref=work_ref.at[slot],
                dst_ref=ptheirs_ref.at[t],
                send_sem=send_sems.at[t],
                recv_sem=recv_sems.at[t],
                device_id=peer,
                device_id_type=pl.DeviceIdType.MESH,
            )

        def send_phase(t, carry):
            slot = lax.rem(t, 2)

            @pl.when(t >= 2)
            def _():
                rdma_for(t - 2, slot).wait_send()

            work_ref[slot] = jnp.dot(
                a_ref[...], b_ref[:, pl.ds(t * TN, TN)],
                preferred_element_type=jnp.float32,
            ).astype(jnp.bfloat16)
            rdma_for(t, slot).start()
            return carry

        lax.fori_loop(0, T, send_phase, 0)
        rdma_for(T - 2, 0).wait_send()
        rdma_for(T - 1, 1).wait_send()

        def recv_phase(t, carry):
            sl = pl.ds(t * TN, TN)
            work_ref[0] = jnp.dot(
                a_ref[...], b_ref[:, sl],
                preferred_element_type=jnp.float32,
            ).astype(jnp.bfloat16)
            rdma_for(t, 0).wait_recv()
            load = pltpu.make_async_copy(
                ptheirs_ref.at[t], work_ref.at[1], copy_sems.at[0])
            load.start()
            load.wait()
            work_ref[0] = (
                work_ref[0].astype(jnp.float32)
                + work_ref[1].astype(jnp.float32)
            ).astype(jnp.bfloat16)
            store = pltpu.make_async_copy(
                work_ref.at[0], out_ref.at[:, sl], copy_sems.at[1])
            store.start()
            store.wait()
            return carry

        lax.fori_loop(0, T, recv_phase, 0)

    out, _ = pl.pallas_call(
        body,
        out_shape=[
            jax.ShapeDtypeStruct((M, N), jnp.bfloat16),
            jax.ShapeDtypeStruct((T, M, TN), jnp.bfloat16),
        ],
        in_specs=[
            pl.BlockSpec(memory_space=pltpu.MemorySpace.VMEM),
            pl.BlockSpec(memory_space=pltpu.MemorySpace.VMEM),
        ],
        out_specs=[
            pl.BlockSpec(memory_space=pl.ANY),
            pl.BlockSpec(memory_space=pl.ANY),
        ],
        scratch_shapes=[
            pltpu.VMEM((2, M, TN), jnp.bfloat16),
            pltpu.SemaphoreType.DMA((T,)),
            pltpu.SemaphoreType.DMA((T,)),
            pltpu.SemaphoreType.DMA((2,)),
        ],
        compiler_params=pltpu.CompilerParams(
            collective_id=0,
            vmem_limit_bytes=64 * 1024 * 1024,
        ),
    )(a, b)
    return out


# device time: 436941 ns/iter; 1.2818x vs baseline; 1.2677x over previous
import jax
import jax.numpy as jnp
from jax import lax
from jax.experimental import pallas as pl
from jax.experimental.pallas import tpu as pltpu

T = 16
NSLOT = 4


def kernel(A, B):
    M, K = A.shape
    _, N = B.shape
    TN = N // T

    a = A.astype(jnp.bfloat16)
    b = B.astype(jnp.bfloat16)

    def body(a_ref, b_ref, out_ref, ptheirs_ref, work_ref, recv_buf,
             send_sems, recv_sems, copy_sems):
        my_x = lax.axis_index("x")
        my_y = lax.axis_index("y")
        peer = (1 - my_x, my_y)

        barrier = pltpu.get_barrier_semaphore()
        pl.semaphore_signal(barrier, inc=1, device_id=peer,
                            device_id_type=pl.DeviceIdType.MESH)
        pl.semaphore_wait(barrier, 1)

        def rdma_for(t, slot):
            return pltpu.make_async_remote_copy(
                src_ref=work_ref.at[slot],
                dst_ref=ptheirs_ref.at[t],
                send_sem=send_sems.at[t],
                recv_sem=recv_sems.at[t],
                device_id=peer,
                device_id_type=pl.DeviceIdType.MESH,
            )

        def step(t, carry):
            @pl.when(t < T)
            def _():
                slot = lax.rem(t, NSLOT)

                @pl.when(t >= NSLOT)
                def _():
                    rdma_for(t - NSLOT, slot).wait_send()

                work_ref[slot] = jnp.dot(
                    a_ref[...], b_ref[:, pl.ds(t * TN, TN)],
                    preferred_element_type=jnp.float32,
                ).astype(jnp.bfloat16)
                rdma_for(t, slot).start()

            @pl.when(t >= 2)
            def _():
                u = t - 2
                uslot = lax.rem(u, NSLOT)
                rdma_for(u, uslot).wait_recv()
                load = pltpu.make_async_copy(
                    ptheirs_ref.at[u], recv_buf, copy_sems.at[0])
                load.start()
                load.wait()
                recv_buf[...] = (
                    work_ref[uslot].astype(jnp.float32)
                    + recv_buf[...].astype(jnp.float32)
                ).astype(jnp.bfloat16)
                store = pltpu.make_async_copy(
                    recv_buf, out_ref.at[:, pl.ds(u * TN, TN)],
                    copy_sems.at[1])
                store.start()
                store.wait()

            return carry

        lax.fori_loop(0, T + 2, step, 0)
        for t in range(T - NSLOT, T):
            rdma_for(t, t % NSLOT).wait_send()

    out, _ = pl.pallas_call(
        body,
        out_shape=[
            jax.ShapeDtypeStruct((M, N), jnp.bfloat16),
            jax.ShapeDtypeStruct((T, M, TN), jnp.bfloat16),
        ],
        in_specs=[
            pl.BlockSpec(memory_space=pltpu.MemorySpace.VMEM),
            pl.BlockSpec(memory_space=pltpu.MemorySpace.VMEM),
        ],
        out_specs=[
            pl.BlockSpec(memory_space=pl.ANY),
            pl.BlockSpec(memory_space=pl.ANY),
        ],
        scratch_shapes=[
            pltpu.VMEM((NSLOT, M, TN), jnp.bfloat16),
            pltpu.VMEM((M, TN), jnp.bfloat16),
            pltpu.SemaphoreType.DMA((T,)),
            pltpu.SemaphoreType.DMA((T,)),
            pltpu.SemaphoreType.DMA((2,)),
        ],
        compiler_params=pltpu.CompilerParams(
            collective_id=0,
            vmem_limit_bytes=64 * 1024 * 1024,
        ),
    )(a, b)
    return out
